# baseline (speedup 1.0000x reference)
import functools

import jax
import jax.numpy as jnp
from jax import lax
from jax.experimental import pallas as pl
from jax.experimental.pallas import tpu as pltpu
from jax.experimental.pallas import tpu_sc as plsc

VOCAB = 1000000
DIM = 128
BATCH = 4096

_info = plsc.get_sparse_core_info()
_NC, _NS, _NL = _info.num_cores, _info.num_subcores, _info.num_lanes
_NW = _NC * _NS
_BPW = BATCH // _NW
_HALF = _BPW // 2

_mesh = plsc.VectorSubcoreMesh(core_axis_name="c", subcore_axis_name="s")


@functools.partial(
    pl.kernel,
    mesh=_mesh,
    compiler_params=pltpu.CompilerParams(
        needs_layout_passes=False,
        skip_device_barrier=True,
        disable_bounds_checks=True,
        disable_semaphore_checks=True,
    ),
    out_type=jax.ShapeDtypeStruct((BATCH, DIM), jnp.float32),
    scratch_types=[
        pltpu.VMEM((_BPW,), jnp.int32),
        pltpu.VMEM((_BPW, DIM), jnp.float32),
        pltpu.SemaphoreType.DMA,
        pltpu.SemaphoreType.DMA,
        pltpu.SemaphoreType.DMA,
    ],
)
def _gather_rows(table_hbm, idx_hbm, out_hbm, idx_v, rows_v, g0, g1, osem):
    wid = lax.axis_index("s") * _NC + lax.axis_index("c")
    base = wid * _BPW
    pltpu.sync_copy(idx_hbm.at[pl.ds(base, _HALF)], idx_v.at[pl.ds(0, _HALF)])
    h0 = pltpu.async_copy(
        table_hbm.at[idx_v.at[pl.ds(0, _HALF)]], rows_v.at[pl.ds(0, _HALF)], g0)
    pltpu.sync_copy(
        idx_hbm.at[pl.ds(base + _HALF, _HALF)], idx_v.at[pl.ds(_HALF, _HALF)])
    h1 = pltpu.async_copy(
        table_hbm.at[idx_v.at[pl.ds(_HALF, _HALF)]],
        rows_v.at[pl.ds(_HALF, _HALF)], g1)
    h0.wait()
    o0 = pltpu.async_copy(
        rows_v.at[pl.ds(0, _HALF)], out_hbm.at[pl.ds(base, _HALF)], osem)
    h1.wait()
    o1 = pltpu.async_copy(
        rows_v.at[pl.ds(_HALF, _HALF)],
        out_hbm.at[pl.ds(base + _HALF, _HALF)], osem)
    o0.wait()
    o1.wait()


def kernel(tokens, embed_table):
    B, T = tokens.shape
    idx = tokens.reshape(B).astype(jnp.int32)
    rows = _gather_rows(embed_table, idx)
    return rows.astype(jnp.bfloat16).reshape(B, T, DIM)


# speedup vs baseline: 1.0139x; 1.0139x over previous
import functools

import jax
import jax.numpy as jnp
from jax import lax
from jax.experimental import pallas as pl
from jax.experimental.pallas import tpu as pltpu
from jax.experimental.pallas import tpu_sc as plsc

VOCAB = 1000000
DIM = 128
BATCH = 4096

_info = plsc.get_sparse_core_info()
_NC, _NS, _NL = _info.num_cores, _info.num_subcores, _info.num_lanes
_NW = _NC * _NS
_BPW = BATCH // _NW
_HALF = _BPW // 2

_mesh = plsc.VectorSubcoreMesh(core_axis_name="c", subcore_axis_name="s")


@functools.partial(
    pl.kernel,
    mesh=_mesh,
    compiler_params=pltpu.CompilerParams(
        needs_layout_passes=False,
        skip_device_barrier=True,
        disable_bounds_checks=True,
        disable_semaphore_checks=True,
    ),
    out_type=jax.ShapeDtypeStruct((BATCH, DIM), jnp.float32),
    scratch_types=[
        pltpu.VMEM((_BPW,), jnp.int32),
        pltpu.VMEM((_BPW, DIM), jnp.float32),
        pltpu.SemaphoreType.DMA,
        pltpu.SemaphoreType.DMA,
        pltpu.SemaphoreType.DMA,
    ],
)
def _gather_rows(table_hbm, idx_hbm, out_hbm, idx_v, rows_v, g0, g1, osem):
    wid = lax.axis_index("s") * _NC + lax.axis_index("c")
    base = wid * _BPW
    pltpu.sync_copy(idx_hbm.at[pl.ds(base, _BPW)], idx_v)
    h0 = pltpu.async_copy(
        table_hbm.at[idx_v.at[pl.ds(0, _HALF)]], rows_v.at[pl.ds(0, _HALF)], g0)
    h1 = pltpu.async_copy(
        table_hbm.at[idx_v.at[pl.ds(_HALF, _HALF)]],
        rows_v.at[pl.ds(_HALF, _HALF)], g1)
    h0.wait()
    o0 = pltpu.async_copy(
        rows_v.at[pl.ds(0, _HALF)], out_hbm.at[pl.ds(base, _HALF)], osem)
    h1.wait()
    o1 = pltpu.async_copy(
        rows_v.at[pl.ds(_HALF, _HALF)],
        out_hbm.at[pl.ds(base + _HALF, _HALF)], osem)
    o0.wait()
    o1.wait()


def kernel(tokens, embed_table):
    B, T = tokens.shape
    idx = tokens.reshape(B).astype(jnp.int32)
    rows = _gather_rows(embed_table, idx)
    return rows.astype(jnp.bfloat16).reshape(B, T, DIM)
